# baseline (speedup 1.0000x reference)
import functools

import jax
import jax.numpy as jnp
from jax import lax
from jax.experimental import pallas as pl
from jax.experimental.pallas import tpu as pltpu
from jax.experimental.pallas import tpu_sc as plsc

_B, _S, _D = 4, 4096, 1024
_NW = 32
_SEQ_PER_W = _S // _NW
_CH = 32
_NCH = _SEQ_PER_W // _CH
_CW = _CH * _D
_UNROLL = 8

_mesh = plsc.VectorSubcoreMesh(core_axis_name="c", subcore_axis_name="s")


@functools.partial(
    pl.kernel,
    mesh=_mesh,
    out_type=jax.ShapeDtypeStruct((_B * _S * _D,), jnp.float32),
    scratch_types=[
        pltpu.VMEM((_CW,), jnp.float32),
        pltpu.VMEM((_CW,), jnp.float32),
    ],
)
def _sc_add(x_hbm, pos_hbm, out_hbm, xbuf, pbuf):
    cid = lax.axis_index("c")
    sid = lax.axis_index("s")
    wid = sid * 2 + cid
    seq0 = wid * _SEQ_PER_W

    def chunk_body(i, carry):
        base = seq0 + i * _CH
        pltpu.sync_copy(pos_hbm.at[pl.ds(base * _D, _CW)], pbuf)

        def batch_body(b, c2):
            w0 = (b * _S + base) * _D
            pltpu.sync_copy(x_hbm.at[pl.ds(w0, _CW)], xbuf)

            def add_body(k, c3):
                for j in range(_UNROLL):
                    sl = pl.ds((k * _UNROLL + j) * 16, 16)
                    plsc.addupdate(xbuf.at[sl], pbuf[sl])
                return c3

            lax.fori_loop(0, _CW // (16 * _UNROLL), add_body, 0)
            pltpu.sync_copy(xbuf, out_hbm.at[pl.ds(w0, _CW)])
            return c2

        return lax.fori_loop(0, _B, batch_body, carry)

    lax.fori_loop(0, _NCH, chunk_body, 0)


def kernel(x, pos_table):
    B, S, D = x.shape
    y = _sc_add(x.reshape(-1), pos_table.reshape(-1))
    return y.reshape(B, S, D)


# speedup vs baseline: 5.5650x; 5.5650x over previous
import jax
import jax.numpy as jnp
from jax.experimental import pallas as pl
from jax.experimental.pallas import tpu as pltpu

_B, _S, _D = 4, 4096, 1024
_CH = 1024
_NCH = _B * _S // _CH
_NBUF = 4


def _pipe_kernel(x_hbm, pos_hbm, out_hbm, xbuf, obuf, pbuf, xsem, osem, psem):
    for q in range(4):
        pltpu.make_async_copy(
            pos_hbm.at[pl.ds(q * _CH, _CH), :],
            pbuf.at[pl.ds(q * _CH, _CH), :],
            psem.at[q],
        ).start()
    for c in range(_NBUF):
        pltpu.make_async_copy(
            x_hbm.at[pl.ds(c * _CH, _CH), :], xbuf.at[c], xsem.at[c]
        ).start()

    for c in range(_NCH):
        slot = c % _NBUF
        q = c % 4
        pltpu.make_async_copy(
            x_hbm.at[pl.ds(c * _CH, _CH), :], xbuf.at[slot], xsem.at[slot]
        ).wait()
        if c < 4:
            pltpu.make_async_copy(
                pos_hbm.at[pl.ds(q * _CH, _CH), :],
                pbuf.at[pl.ds(q * _CH, _CH), :],
                psem.at[q],
            ).wait()
        if c >= _NBUF:
            pltpu.make_async_copy(
                obuf.at[slot],
                out_hbm.at[pl.ds((c - _NBUF) * _CH, _CH), :],
                osem.at[slot],
            ).wait()
        obuf[slot] = xbuf[slot] + pbuf[pl.ds(q * _CH, _CH), :]
        pltpu.make_async_copy(
            obuf.at[slot], out_hbm.at[pl.ds(c * _CH, _CH), :], osem.at[slot]
        ).start()
        nxt = c + _NBUF
        if nxt < _NCH:
            pltpu.make_async_copy(
                x_hbm.at[pl.ds(nxt * _CH, _CH), :], xbuf.at[slot], xsem.at[slot]
            ).start()

    for c in range(_NCH - _NBUF, _NCH):
        slot = c % _NBUF
        pltpu.make_async_copy(
            obuf.at[slot], out_hbm.at[pl.ds(c * _CH, _CH), :], osem.at[slot]
        ).wait()


def kernel(x, pos_table):
    B, S, D = x.shape
    y = pl.pallas_call(
        _pipe_kernel,
        in_specs=[
            pl.BlockSpec(memory_space=pltpu.MemorySpace.HBM),
            pl.BlockSpec(memory_space=pltpu.MemorySpace.HBM),
        ],
        out_specs=pl.BlockSpec(memory_space=pltpu.MemorySpace.HBM),
        out_shape=jax.ShapeDtypeStruct((B * S, D), x.dtype),
        scratch_shapes=[
            pltpu.VMEM((_NBUF, _CH, _D), jnp.float32),
            pltpu.VMEM((_NBUF, _CH, _D), jnp.float32),
            pltpu.VMEM((_S, _D), jnp.float32),
            pltpu.SemaphoreType.DMA((_NBUF,)),
            pltpu.SemaphoreType.DMA((_NBUF,)),
            pltpu.SemaphoreType.DMA((4,)),
        ],
    )(x.reshape(B * S, D), pos_table)
    return y.reshape(B, S, D)
